# baseline (speedup 1.0000x reference)
import functools

import jax
import jax.numpy as jnp
from jax import lax
from jax.experimental import pallas as pl
from jax.experimental.pallas import tpu as pltpu
from jax.experimental.pallas import tpu_sc as plsc

N = 10000
E = 320000
K = 128
EP = 327680
PAD = EP - E
NCH = EP // K
NC, NS = 2, 16
CS = NCH // (NC * NS)
DEGW = 10240
RPS = 624
TAIL = N - NS * RPS

_MESH = dict(core_axis_name="c", subcore_axis_name="s", num_cores=NC,
             num_subcores=NS)
_CPARAMS = pltpu.CompilerParams(needs_layout_passes=False)


_DG = 40


def _deg_body(dst_hbm, out_hbm, hist, dbuf, red, outv, hist_sh):
    c = lax.axis_index("c")
    s = lax.axis_index("s")
    wid = c * NS + s

    zz = jnp.zeros((16,), jnp.float32)

    def zero(i, carry):
        hist[pl.ds(i * 16, 16)] = zz
        return carry

    lax.fori_loop(0, DEGW // 16, zero, 0)

    ones = jnp.ones((16,), jnp.float32)
    base = wid * CS

    def group(g, carry):
        pltpu.sync_copy(dst_hbm.at[pl.ds(base + g * _DG, _DG)], dbuf)

        def body(j, carry2):
            for t in range(K // 16):
                idx = dbuf[j, pl.ds(t * 16, 16)]
                plsc.addupdate_scatter(hist, [idx], ones)
            return carry2

        return lax.fori_loop(0, _DG, body, carry)

    lax.fori_loop(0, CS // _DG, group, 0)

    pltpu.sync_copy(hist, hist_sh.at[s])
    plsc.subcore_barrier()
    for r in range(NS):
        pltpu.sync_copy(hist_sh.at[r, pl.ds(s * 640, 640)], red.at[r])

    def reduce(j, carry):
        acc = red[0, pl.ds(j * 16, 16)]
        for r in range(1, NS):
            acc = acc + red[r, pl.ds(j * 16, 16)]
        outv[pl.ds(j * 16, 16)] = acc
        return carry

    lax.fori_loop(0, 640 // 16, reduce, 0)
    pltpu.sync_copy(outv, out_hbm.at[pl.ds(c * DEGW + s * 640, 640)])


def _deg_call(dst2d):
    return pl.kernel(
        _deg_body,
        out_type=jax.ShapeDtypeStruct((NC * DEGW,), jnp.float32),
        mesh=plsc.VectorSubcoreMesh(**_MESH),
        compiler_params=_CPARAMS,
        scratch_types=[
            pltpu.VMEM((DEGW,), jnp.float32),
            pltpu.VMEM((_DG, K), jnp.int32),
            pltpu.VMEM((NS, 640), jnp.float32),
            pltpu.VMEM((640,), jnp.float32),
            pltpu.VMEM_SHARED((NS, DEGW), jnp.float32),
        ],
    )(dst2d)


_NBUF = 2
_G = 40


def _agg_body(v_hbm, s_hbm, d_hbm, z_hbm, out_hbm,
              sbuf, dbuf, rows, sems, acc):
    c = lax.axis_index("c")
    s = lax.axis_index("s")
    row0 = (c * NS + s) * CS

    pltpu.sync_copy(z_hbm, acc.at[pl.ds(s * RPS, RPS)])

    @pl.when(s == NS - 1)
    def _zero_tail():
        pltpu.sync_copy(z_hbm.at[pl.ds(0, TAIL)],
                        acc.at[pl.ds(NS * RPS, TAIL)])

    plsc.subcore_barrier()

    def group(g, carry):
        pltpu.sync_copy(s_hbm.at[pl.ds(row0 + g * _G, _G)], sbuf)
        pltpu.sync_copy(d_hbm.at[pl.ds(row0 + g * _G, _G)], dbuf)
        for b in range(_NBUF):
            pltpu.async_copy(v_hbm.at[sbuf.at[b]], rows.at[b], sems.at[b])
        for t in range(_G):
            b = t % _NBUF
            pltpu.make_async_copy(v_hbm.at[sbuf.at[b]], rows.at[b],
                                  sems.at[b]).wait()
            pltpu.sync_copy(rows.at[b], acc.at[dbuf.at[t]], add=True)
            if t + _NBUF < _G:
                pltpu.async_copy(v_hbm.at[sbuf.at[t + _NBUF]], rows.at[b],
                                 sems.at[b])
        return carry

    lax.fori_loop(0, CS // _G, group, 0)
    plsc.subcore_barrier()

    pltpu.sync_copy(acc.at[pl.ds(s * RPS, RPS)],
                    out_hbm.at[pl.ds(c * N + s * RPS, RPS)])

    @pl.when(s == NS - 1)
    def _copy_tail():
        pltpu.sync_copy(acc.at[pl.ds(NS * RPS, TAIL)],
                        out_hbm.at[pl.ds(c * N + NS * RPS, TAIL)])


def _agg_call(vrows, src2d, dst2d, zeros):
    return pl.kernel(
        _agg_body,
        out_type=jax.ShapeDtypeStruct((NC * N, 128), jnp.float32),
        mesh=plsc.VectorSubcoreMesh(**_MESH),
        compiler_params=_CPARAMS,
        scratch_types=[
            pltpu.VMEM((_G, K), jnp.int32),
            pltpu.VMEM((_G, K), jnp.int32),
            pltpu.VMEM((_NBUF, K, 128), jnp.float32),
            pltpu.SemaphoreType.DMA((_NBUF,)),
            pltpu.VMEM_SHARED((N + 8, 128), jnp.float32),
        ],
    )(vrows, src2d, dst2d, zeros)


_BN = 2000


def _dis(dg_ref):
    return lax.rsqrt(dg_ref[:, 0:1] + dg_ref[:, 1:2] + 1.0)


def _tca_body(x_ref, dg_ref, o_ref):
    o_ref[...] = x_ref[...] * _dis(dg_ref)


def _tca(x, degT):
    return pl.pallas_call(
        _tca_body,
        grid=(N // _BN,),
        in_specs=[
            pl.BlockSpec((_BN, 128), lambda i: (i, 0)),
            pl.BlockSpec((_BN, 2), lambda i: (i, 0)),
        ],
        out_specs=pl.BlockSpec((_BN, 128), lambda i: (i, 0)),
        out_shape=jax.ShapeDtypeStruct((N, 128), jnp.float32),
    )(x, degT)


def _tcb_body(u_ref, xh_ref, dg_ref, w1_ref, b1_ref, w2_ref, o_ref):
    dis = _dis(dg_ref)
    y = u_ref[0] + u_ref[1] + xh_ref[...]
    h = jnp.dot(y, w1_ref[...], preferred_element_type=jnp.float32)
    z = jnp.maximum(h * dis + b1_ref[...], 0.0)
    o_ref[...] = jnp.dot(z * dis, w2_ref[...],
                         preferred_element_type=jnp.float32)


def _tcb(u, xh, degT, W1, b1r, W2):
    return pl.pallas_call(
        _tcb_body,
        grid=(N // _BN,),
        in_specs=[
            pl.BlockSpec((2, _BN, 128), lambda i: (0, i, 0)),
            pl.BlockSpec((_BN, 128), lambda i: (i, 0)),
            pl.BlockSpec((_BN, 2), lambda i: (i, 0)),
            pl.BlockSpec((128, 256), lambda i: (0, 0)),
            pl.BlockSpec((1, 256), lambda i: (0, 0)),
            pl.BlockSpec((256, 128), lambda i: (0, 0)),
        ],
        out_specs=pl.BlockSpec((_BN, 128), lambda i: (i, 0)),
        out_shape=jax.ShapeDtypeStruct((N, 128), jnp.float32),
    )(u, xh, degT, W1, b1r, W2)


def _tcc_body(s_ref, v_ref, dg_ref, b_ref, o_ref):
    o_ref[...] = ((s_ref[0] + s_ref[1] + v_ref[...]) * _dis(dg_ref)
                  + b_ref[...])


def _tcc(s2, v2, degT, b2r):
    return pl.pallas_call(
        _tcc_body,
        grid=(N // _BN,),
        in_specs=[
            pl.BlockSpec((2, _BN, 128), lambda i: (0, i, 0)),
            pl.BlockSpec((_BN, 128), lambda i: (i, 0)),
            pl.BlockSpec((_BN, 2), lambda i: (i, 0)),
            pl.BlockSpec((1, 128), lambda i: (0, 0)),
        ],
        out_specs=pl.BlockSpec((_BN, 128), lambda i: (i, 0)),
        out_shape=jax.ShapeDtypeStruct((N, 128), jnp.float32),
    )(s2, v2, degT, b2r)


def kernel(x, edge_index, W1, b1, W2, b2):
    ei = edge_index.astype(jnp.int32)
    src2d = jnp.concatenate([ei[0], jnp.zeros((PAD,), jnp.int32)]
                            ).reshape(NCH, K)
    pad_dst = N + (jnp.arange(PAD, dtype=jnp.int32) % 8)
    dst2d = jnp.concatenate([ei[1], pad_dst]).reshape(NCH, K)
    zeros = jnp.zeros((RPS, 128), jnp.float32)

    degT = _deg_call(dst2d).reshape(NC, DEGW).T
    xh = _tca(x, degT)
    u = _agg_call(xh, src2d, dst2d, zeros)
    v2 = _tcb(u.reshape(2, N, 128), xh, degT,
              W1, b1.reshape(1, 256), W2)
    s2 = _agg_call(v2, src2d, dst2d, zeros)
    s2b = _agg_call(v2 * 0.0, src2d, dst2d, zeros)
    return _tcc((s2 + s2b).reshape(2, N, 128), v2, degT, b2.reshape(1, 128))


# speedup vs baseline: 1.6089x; 1.6089x over previous
import functools

import jax
import jax.numpy as jnp
from jax import lax
from jax.experimental import pallas as pl
from jax.experimental.pallas import tpu as pltpu
from jax.experimental.pallas import tpu_sc as plsc

N = 10000
E = 320000
K = 128
EP = 327680
PAD = EP - E
NCH = EP // K
NC, NS = 2, 16
CS = NCH // (NC * NS)
DEGW = 10240
RPS = 624
TAIL = N - NS * RPS

_MESH = dict(core_axis_name="c", subcore_axis_name="s", num_cores=NC,
             num_subcores=NS)
_CPARAMS = pltpu.CompilerParams(needs_layout_passes=False)


_DG = 40


def _deg_body(dst_hbm, out_hbm, hist, dbuf, red, outv, hist_sh):
    c = lax.axis_index("c")
    s = lax.axis_index("s")
    wid = c * NS + s

    zz = jnp.zeros((16,), jnp.float32)

    def zero(i, carry):
        hist[pl.ds(i * 16, 16)] = zz
        return carry

    lax.fori_loop(0, DEGW // 16, zero, 0)

    ones = jnp.ones((16,), jnp.float32)
    base = wid * CS

    def group(g, carry):
        pltpu.sync_copy(dst_hbm.at[pl.ds(base + g * _DG, _DG)], dbuf)

        def body(j, carry2):
            for t in range(K // 16):
                idx = dbuf[j, pl.ds(t * 16, 16)]
                plsc.addupdate_scatter(hist, [idx], ones)
            return carry2

        return lax.fori_loop(0, _DG, body, carry)

    lax.fori_loop(0, CS // _DG, group, 0)

    pltpu.sync_copy(hist, hist_sh.at[s])
    plsc.subcore_barrier()
    for r in range(NS):
        pltpu.sync_copy(hist_sh.at[r, pl.ds(s * 640, 640)], red.at[r])

    def reduce(j, carry):
        acc = red[0, pl.ds(j * 16, 16)]
        for r in range(1, NS):
            acc = acc + red[r, pl.ds(j * 16, 16)]
        outv[pl.ds(j * 16, 16)] = acc
        return carry

    lax.fori_loop(0, 640 // 16, reduce, 0)
    pltpu.sync_copy(outv, out_hbm.at[pl.ds(c * DEGW + s * 640, 640)])


def _deg_call(dst2d):
    return pl.kernel(
        _deg_body,
        out_type=jax.ShapeDtypeStruct((NC * DEGW,), jnp.float32),
        mesh=plsc.VectorSubcoreMesh(**_MESH),
        compiler_params=_CPARAMS,
        scratch_types=[
            pltpu.VMEM((DEGW,), jnp.float32),
            pltpu.VMEM((_DG, K), jnp.int32),
            pltpu.VMEM((NS, 640), jnp.float32),
            pltpu.VMEM((640,), jnp.float32),
            pltpu.VMEM_SHARED((NS, DEGW), jnp.float32),
        ],
    )(dst2d)


_NBUF = 2
_G = 40


def _agg_body(v_hbm, s_hbm, d_hbm, z_hbm, out_hbm,
              sbuf, dbuf, rows, sems, ssems, acc):
    c = lax.axis_index("c")
    s = lax.axis_index("s")
    row0 = (c * NS + s) * CS

    pltpu.sync_copy(z_hbm, acc.at[pl.ds(s * RPS, RPS)])

    @pl.when(s == NS - 1)
    def _zero_tail():
        pltpu.sync_copy(z_hbm.at[pl.ds(0, TAIL)],
                        acc.at[pl.ds(NS * RPS, TAIL)])

    plsc.subcore_barrier()

    def group(g, carry):
        pltpu.sync_copy(s_hbm.at[pl.ds(row0 + g * _G, _G)], sbuf)
        pltpu.sync_copy(d_hbm.at[pl.ds(row0 + g * _G, _G)], dbuf)
        pltpu.async_copy(v_hbm.at[sbuf.at[0]], rows.at[0], sems.at[0])
        for t in range(_G):
            b = t % _NBUF
            o = (t + 1) % _NBUF
            pltpu.make_async_copy(v_hbm.at[sbuf.at[b]], rows.at[b],
                                  sems.at[b]).wait()
            pltpu.async_copy(rows.at[b], acc.at[dbuf.at[t]], ssems.at[b],
                             add=True)
            if t > 0:
                pltpu.make_async_copy(rows.at[o], acc.at[dbuf.at[t - 1]],
                                      ssems.at[o]).wait()
            if t + 1 < _G:
                pltpu.async_copy(v_hbm.at[sbuf.at[t + 1]], rows.at[o],
                                 sems.at[o])
        pltpu.make_async_copy(rows.at[(_G - 1) % _NBUF],
                              acc.at[dbuf.at[_G - 1]],
                              ssems.at[(_G - 1) % _NBUF]).wait()
        return carry

    lax.fori_loop(0, CS // _G, group, 0)
    plsc.subcore_barrier()

    pltpu.sync_copy(acc.at[pl.ds(s * RPS, RPS)],
                    out_hbm.at[pl.ds(c * N + s * RPS, RPS)])

    @pl.when(s == NS - 1)
    def _copy_tail():
        pltpu.sync_copy(acc.at[pl.ds(NS * RPS, TAIL)],
                        out_hbm.at[pl.ds(c * N + NS * RPS, TAIL)])


def _agg_call(vrows, src2d, dst2d, zeros):
    return pl.kernel(
        _agg_body,
        out_type=jax.ShapeDtypeStruct((NC * N, 128), jnp.float32),
        mesh=plsc.VectorSubcoreMesh(**_MESH),
        compiler_params=_CPARAMS,
        scratch_types=[
            pltpu.VMEM((_G, K), jnp.int32),
            pltpu.VMEM((_G, K), jnp.int32),
            pltpu.VMEM((_NBUF, K, 128), jnp.float32),
            pltpu.SemaphoreType.DMA((_NBUF,)),
            pltpu.SemaphoreType.DMA((_NBUF,)),
            pltpu.VMEM_SHARED((N + 8, 128), jnp.float32),
        ],
    )(vrows, src2d, dst2d, zeros)


_BN = 2000


def _dis(dg_ref):
    return lax.rsqrt(dg_ref[:, 0:1] + dg_ref[:, 1:2] + 1.0)


def _tca_body(x_ref, dg_ref, o_ref):
    o_ref[...] = x_ref[...] * _dis(dg_ref)


def _tca(x, degT):
    return pl.pallas_call(
        _tca_body,
        grid=(N // _BN,),
        in_specs=[
            pl.BlockSpec((_BN, 128), lambda i: (i, 0)),
            pl.BlockSpec((_BN, 2), lambda i: (i, 0)),
        ],
        out_specs=pl.BlockSpec((_BN, 128), lambda i: (i, 0)),
        out_shape=jax.ShapeDtypeStruct((N, 128), jnp.float32),
    )(x, degT)


def _tcb_body(u_ref, xh_ref, dg_ref, w1_ref, b1_ref, w2_ref, o_ref):
    dis = _dis(dg_ref)
    y = u_ref[0] + u_ref[1] + xh_ref[...]
    h = jnp.dot(y, w1_ref[...], preferred_element_type=jnp.float32)
    z = jnp.maximum(h * dis + b1_ref[...], 0.0)
    o_ref[...] = jnp.dot(z * dis, w2_ref[...],
                         preferred_element_type=jnp.float32)


def _tcb(u, xh, degT, W1, b1r, W2):
    return pl.pallas_call(
        _tcb_body,
        grid=(N // _BN,),
        in_specs=[
            pl.BlockSpec((2, _BN, 128), lambda i: (0, i, 0)),
            pl.BlockSpec((_BN, 128), lambda i: (i, 0)),
            pl.BlockSpec((_BN, 2), lambda i: (i, 0)),
            pl.BlockSpec((128, 256), lambda i: (0, 0)),
            pl.BlockSpec((1, 256), lambda i: (0, 0)),
            pl.BlockSpec((256, 128), lambda i: (0, 0)),
        ],
        out_specs=pl.BlockSpec((_BN, 128), lambda i: (i, 0)),
        out_shape=jax.ShapeDtypeStruct((N, 128), jnp.float32),
    )(u, xh, degT, W1, b1r, W2)


def _tcc_body(s_ref, v_ref, dg_ref, b_ref, o_ref):
    o_ref[...] = ((s_ref[0] + s_ref[1] + v_ref[...]) * _dis(dg_ref)
                  + b_ref[...])


def _tcc(s2, v2, degT, b2r):
    return pl.pallas_call(
        _tcc_body,
        grid=(N // _BN,),
        in_specs=[
            pl.BlockSpec((2, _BN, 128), lambda i: (0, i, 0)),
            pl.BlockSpec((_BN, 128), lambda i: (i, 0)),
            pl.BlockSpec((_BN, 2), lambda i: (i, 0)),
            pl.BlockSpec((1, 128), lambda i: (0, 0)),
        ],
        out_specs=pl.BlockSpec((_BN, 128), lambda i: (i, 0)),
        out_shape=jax.ShapeDtypeStruct((N, 128), jnp.float32),
    )(s2, v2, degT, b2r)


def kernel(x, edge_index, W1, b1, W2, b2):
    ei = edge_index.astype(jnp.int32)
    src2d = jnp.concatenate([ei[0], jnp.zeros((PAD,), jnp.int32)]
                            ).reshape(NCH, K)
    pad_dst = N + (jnp.arange(PAD, dtype=jnp.int32) % 8)
    dst2d = jnp.concatenate([ei[1], pad_dst]).reshape(NCH, K)
    zeros = jnp.zeros((RPS, 128), jnp.float32)

    degT = _deg_call(dst2d).reshape(NC, DEGW).T
    xh = _tca(x, degT)
    u = _agg_call(xh, src2d, dst2d, zeros)
    v2 = _tcb(u.reshape(2, N, 128), xh, degT,
              W1, b1.reshape(1, 256), W2)
    s2 = _agg_call(v2, src2d, dst2d, zeros)
    return _tcc(s2.reshape(2, N, 128), v2, degT, b2.reshape(1, 128))


# speedup vs baseline: 1.7252x; 1.0723x over previous
import functools

import jax
import jax.numpy as jnp
from jax import lax
from jax.experimental import pallas as pl
from jax.experimental.pallas import tpu as pltpu
from jax.experimental.pallas import tpu_sc as plsc

N = 10000
E = 320000
K = 128
EP = 327680
PAD = EP - E
NCH = EP // K
NC, NS = 2, 16
CS = NCH // (NC * NS)
DEGW = 10240
RPS = 624
TAIL = N - NS * RPS

_MESH = dict(core_axis_name="c", subcore_axis_name="s", num_cores=NC,
             num_subcores=NS)
_CPARAMS = pltpu.CompilerParams(needs_layout_passes=False)


_DG = 40


def _deg_body(dst_hbm, out_hbm, hist, dbuf, red, outv, hist_sh):
    c = lax.axis_index("c")
    s = lax.axis_index("s")
    wid = c * NS + s

    zz = jnp.zeros((16,), jnp.float32)

    def zero(i, carry):
        hist[pl.ds(i * 16, 16)] = zz
        return carry

    lax.fori_loop(0, DEGW // 16, zero, 0)

    ones = jnp.ones((16,), jnp.float32)
    base = wid * CS

    def group(g, carry):
        pltpu.sync_copy(dst_hbm.at[pl.ds(base + g * _DG, _DG)], dbuf)

        def body(j, carry2):
            for t in range(K // 16):
                idx = dbuf[j, pl.ds(t * 16, 16)]
                plsc.addupdate_scatter(hist, [idx], ones)
            return carry2

        return lax.fori_loop(0, _DG, body, carry)

    lax.fori_loop(0, CS // _DG, group, 0)

    pltpu.sync_copy(hist, hist_sh.at[s])
    plsc.subcore_barrier()
    for r in range(NS):
        pltpu.sync_copy(hist_sh.at[r, pl.ds(s * 640, 640)], red.at[r])

    def reduce(j, carry):
        acc = red[0, pl.ds(j * 16, 16)]
        for r in range(1, NS):
            acc = acc + red[r, pl.ds(j * 16, 16)]
        outv[pl.ds(j * 16, 16)] = acc
        return carry

    lax.fori_loop(0, 640 // 16, reduce, 0)
    pltpu.sync_copy(outv, out_hbm.at[pl.ds(c * DEGW + s * 640, 640)])


def _deg_call(dst2d):
    return pl.kernel(
        _deg_body,
        out_type=jax.ShapeDtypeStruct((NC * DEGW,), jnp.float32),
        mesh=plsc.VectorSubcoreMesh(**_MESH),
        compiler_params=_CPARAMS,
        scratch_types=[
            pltpu.VMEM((DEGW,), jnp.float32),
            pltpu.VMEM((_DG, K), jnp.int32),
            pltpu.VMEM((NS, 640), jnp.float32),
            pltpu.VMEM((640,), jnp.float32),
            pltpu.VMEM_SHARED((NS, DEGW), jnp.float32),
        ],
    )(dst2d)


_NBUF = 2
_G = 40


def _agg_body(v_hbm, s_hbm, d_hbm, out_hbm,
              sbuf, dbuf, rows, sems, ssems, acc):
    c = lax.axis_index("c")
    s = lax.axis_index("s")
    row0 = (c * NS + s) * CS

    zz = jnp.zeros((16,), jnp.float32)

    def _zrow(r, carry):
        for cc in range(8):
            rows[0, r, pl.ds(cc * 16, 16)] = zz
        return carry

    lax.fori_loop(0, K, _zrow, 0)
    for blk in range(RPS // K):
        pltpu.sync_copy(rows.at[0], acc.at[pl.ds(s * RPS + blk * K, K)])
    pltpu.sync_copy(rows.at[0, pl.ds(0, RPS % K)],
                    acc.at[pl.ds(s * RPS + (RPS // K) * K, RPS % K)])

    @pl.when(s == NS - 1)
    def _zero_tail():
        pltpu.sync_copy(rows.at[0, pl.ds(0, TAIL)],
                        acc.at[pl.ds(NS * RPS, TAIL)])

    plsc.subcore_barrier()

    def group(g, carry):
        pltpu.sync_copy(s_hbm.at[pl.ds(row0 + g * _G, _G)], sbuf)
        pltpu.sync_copy(d_hbm.at[pl.ds(row0 + g * _G, _G)], dbuf)
        for b in range(_NBUF):
            pltpu.async_copy(v_hbm.at[sbuf.at[b]], rows.at[b], sems.at[b])
        for t in range(_G):
            b = t % _NBUF
            pltpu.make_async_copy(v_hbm.at[sbuf.at[b]], rows.at[b],
                                  sems.at[b]).wait()
            pltpu.sync_copy(rows.at[b], acc.at[dbuf.at[t]], add=True)
            if t + _NBUF < _G:
                pltpu.async_copy(v_hbm.at[sbuf.at[t + _NBUF]], rows.at[b],
                                 sems.at[b])
        return carry

    lax.fori_loop(0, CS // _G, group, 0)
    plsc.subcore_barrier()

    pltpu.sync_copy(acc.at[pl.ds(s * RPS, RPS)],
                    out_hbm.at[pl.ds(c * N + s * RPS, RPS)])

    @pl.when(s == NS - 1)
    def _copy_tail():
        pltpu.sync_copy(acc.at[pl.ds(NS * RPS, TAIL)],
                        out_hbm.at[pl.ds(c * N + NS * RPS, TAIL)])


def _agg_call(vrows, src2d, dst2d):
    return pl.kernel(
        _agg_body,
        out_type=jax.ShapeDtypeStruct((NC * N, 128), jnp.float32),
        mesh=plsc.VectorSubcoreMesh(**_MESH),
        compiler_params=_CPARAMS,
        scratch_types=[
            pltpu.VMEM((_G, K), jnp.int32),
            pltpu.VMEM((_G, K), jnp.int32),
            pltpu.VMEM((_NBUF, K, 128), jnp.float32),
            pltpu.SemaphoreType.DMA((_NBUF,)),
            pltpu.SemaphoreType.DMA((_NBUF,)),
            pltpu.VMEM_SHARED((N + 8, 128), jnp.float32),
        ],
    )(vrows, src2d, dst2d)


_BN = 2000


def _dis(dg_ref):
    return lax.rsqrt(dg_ref[:, 0:1] + dg_ref[:, 1:2] + 1.0)


def _tca_body(x_ref, dg_ref, o_ref):
    o_ref[...] = x_ref[...] * _dis(dg_ref)


def _tca(x, degT):
    return pl.pallas_call(
        _tca_body,
        grid=(N // _BN,),
        in_specs=[
            pl.BlockSpec((_BN, 128), lambda i: (i, 0)),
            pl.BlockSpec((_BN, 2), lambda i: (i, 0)),
        ],
        out_specs=pl.BlockSpec((_BN, 128), lambda i: (i, 0)),
        out_shape=jax.ShapeDtypeStruct((N, 128), jnp.float32),
    )(x, degT)


def _tcb_body(u_ref, xh_ref, dg_ref, w1_ref, b1_ref, w2_ref, o_ref):
    dis = _dis(dg_ref)
    y = u_ref[0] + u_ref[1] + xh_ref[...]
    h = jnp.dot(y, w1_ref[...], preferred_element_type=jnp.float32)
    z = jnp.maximum(h * dis + b1_ref[...], 0.0)
    o_ref[...] = jnp.dot(z * dis, w2_ref[...],
                         preferred_element_type=jnp.float32)


def _tcb(u, xh, degT, W1, b1r, W2):
    return pl.pallas_call(
        _tcb_body,
        grid=(N // _BN,),
        in_specs=[
            pl.BlockSpec((2, _BN, 128), lambda i: (0, i, 0)),
            pl.BlockSpec((_BN, 128), lambda i: (i, 0)),
            pl.BlockSpec((_BN, 2), lambda i: (i, 0)),
            pl.BlockSpec((128, 256), lambda i: (0, 0)),
            pl.BlockSpec((1, 256), lambda i: (0, 0)),
            pl.BlockSpec((256, 128), lambda i: (0, 0)),
        ],
        out_specs=pl.BlockSpec((_BN, 128), lambda i: (i, 0)),
        out_shape=jax.ShapeDtypeStruct((N, 128), jnp.float32),
    )(u, xh, degT, W1, b1r, W2)


def _tcc_body(s_ref, v_ref, dg_ref, b_ref, o_ref):
    o_ref[...] = ((s_ref[0] + s_ref[1] + v_ref[...]) * _dis(dg_ref)
                  + b_ref[...])


def _tcc(s2, v2, degT, b2r):
    return pl.pallas_call(
        _tcc_body,
        grid=(N // _BN,),
        in_specs=[
            pl.BlockSpec((2, _BN, 128), lambda i: (0, i, 0)),
            pl.BlockSpec((_BN, 128), lambda i: (i, 0)),
            pl.BlockSpec((_BN, 2), lambda i: (i, 0)),
            pl.BlockSpec((1, 128), lambda i: (0, 0)),
        ],
        out_specs=pl.BlockSpec((_BN, 128), lambda i: (i, 0)),
        out_shape=jax.ShapeDtypeStruct((N, 128), jnp.float32),
    )(s2, v2, degT, b2r)


def kernel(x, edge_index, W1, b1, W2, b2):
    ei = edge_index.astype(jnp.int32)
    src2d = jnp.concatenate([ei[0], jnp.zeros((PAD,), jnp.int32)]
                            ).reshape(NCH, K)
    pad_dst = N + (jnp.arange(PAD, dtype=jnp.int32) % 8)
    dst2d = jnp.concatenate([ei[1], pad_dst]).reshape(NCH, K)

    degT = _deg_call(dst2d).reshape(NC, DEGW).T
    xh = _tca(x, degT)
    u = _agg_call(xh, src2d, dst2d)
    v2 = _tcb(u.reshape(2, N, 128), xh, degT,
              W1, b1.reshape(1, 256), W2)
    s2 = _agg_call(v2, src2d, dst2d)
    return _tcc(s2.reshape(2, N, 128), v2, degT, b2.reshape(1, 128))


# speedup vs baseline: 1.7561x; 1.0179x over previous
import functools

import jax
import jax.numpy as jnp
from jax import lax
from jax.experimental import pallas as pl
from jax.experimental.pallas import tpu as pltpu
from jax.experimental.pallas import tpu_sc as plsc

N = 10000
E = 320000
K = 128
EP = 327680
PAD = EP - E
NCH = EP // K
NC, NS = 2, 16
CS = NCH // (NC * NS)
DEGW = 10240
RPS = 624
TAIL = N - NS * RPS

_MESH = dict(core_axis_name="c", subcore_axis_name="s", num_cores=NC,
             num_subcores=NS)
_CPARAMS = pltpu.CompilerParams(needs_layout_passes=False)


_DG = 40


def _deg_body(dst_hbm, out_hbm, hist, dbuf, red, outv, hist_sh):
    c = lax.axis_index("c")
    s = lax.axis_index("s")
    wid = c * NS + s

    zz = jnp.zeros((16,), jnp.float32)

    def zero(i, carry):
        hist[pl.ds(i * 16, 16)] = zz
        return carry

    lax.fori_loop(0, DEGW // 16, zero, 0)

    ones = jnp.ones((16,), jnp.float32)
    base = wid * CS

    def group(g, carry):
        pltpu.sync_copy(dst_hbm.at[pl.ds(base + g * _DG, _DG)], dbuf)

        def body(j, carry2):
            for t in range(K // 16):
                idx = dbuf[j, pl.ds(t * 16, 16)]
                plsc.addupdate_scatter(hist, [idx], ones)
            return carry2

        return lax.fori_loop(0, _DG, body, carry)

    lax.fori_loop(0, CS // _DG, group, 0)

    pltpu.sync_copy(hist, hist_sh.at[s])
    plsc.subcore_barrier()
    for r in range(NS):
        pltpu.sync_copy(hist_sh.at[r, pl.ds(s * 640, 640)], red.at[r])

    def reduce(j, carry):
        acc = red[0, pl.ds(j * 16, 16)]
        for r in range(1, NS):
            acc = acc + red[r, pl.ds(j * 16, 16)]
        outv[pl.ds(j * 16, 16)] = acc
        return carry

    lax.fori_loop(0, 640 // 16, reduce, 0)
    pltpu.sync_copy(outv, out_hbm.at[pl.ds(c * DEGW + s * 640, 640)])


def _deg_call(dst2d):
    return pl.kernel(
        _deg_body,
        out_type=jax.ShapeDtypeStruct((NC * DEGW,), jnp.float32),
        mesh=plsc.VectorSubcoreMesh(**_MESH),
        compiler_params=_CPARAMS,
        scratch_types=[
            pltpu.VMEM((DEGW,), jnp.float32),
            pltpu.VMEM((_DG, K), jnp.int32),
            pltpu.VMEM((NS, 640), jnp.float32),
            pltpu.VMEM((640,), jnp.float32),
            pltpu.VMEM_SHARED((NS, DEGW), jnp.float32),
        ],
    )(dst2d)


_NBUF = 2
_G = 40


def _agg_body(v_hbm, s_hbm, d_hbm, out_hbm,
              sbuf, dbuf, rows, sems, ssems, acc):
    c = lax.axis_index("c")
    s = lax.axis_index("s")
    row0 = (c * NS + s) * CS

    zz = jnp.zeros((16,), jnp.float32)

    def _zrow(r, carry):
        for cc in range(8):
            rows[0, r, pl.ds(cc * 16, 16)] = zz
        return carry

    lax.fori_loop(0, K, _zrow, 0)
    for blk in range(RPS // K):
        pltpu.sync_copy(rows.at[0], acc.at[pl.ds(s * RPS + blk * K, K)])
    pltpu.sync_copy(rows.at[0, pl.ds(0, RPS % K)],
                    acc.at[pl.ds(s * RPS + (RPS // K) * K, RPS % K)])

    @pl.when(s == NS - 1)
    def _zero_tail():
        pltpu.sync_copy(rows.at[0, pl.ds(0, TAIL)],
                        acc.at[pl.ds(NS * RPS, TAIL)])

    plsc.subcore_barrier()

    def group(g, carry):
        pltpu.sync_copy(s_hbm.at[pl.ds(row0 + g * _G, _G)], sbuf)
        pltpu.sync_copy(d_hbm.at[pl.ds(row0 + g * _G, _G)], dbuf)
        for b in range(_NBUF):
            pltpu.async_copy(v_hbm.at[sbuf.at[b]], rows.at[b], sems.at[b])
        for t in range(_G):
            b = t % _NBUF
            pltpu.make_async_copy(v_hbm.at[sbuf.at[b]], rows.at[b],
                                  sems.at[b]).wait()
            pltpu.sync_copy(rows.at[b], acc.at[dbuf.at[t]], add=True)
            if t + _NBUF < _G:
                pltpu.async_copy(v_hbm.at[sbuf.at[t + _NBUF]], rows.at[b],
                                 sems.at[b])
        return carry

    lax.fori_loop(0, CS // _G, group, 0)
    plsc.subcore_barrier()

    pltpu.sync_copy(acc.at[pl.ds(s * RPS, RPS)],
                    out_hbm.at[pl.ds(c * N + s * RPS, RPS)])

    @pl.when(s == NS - 1)
    def _copy_tail():
        pltpu.sync_copy(acc.at[pl.ds(NS * RPS, TAIL)],
                        out_hbm.at[pl.ds(c * N + NS * RPS, TAIL)])


def _agg_call(vrows, src2d, dst2d):
    return pl.kernel(
        _agg_body,
        out_type=jax.ShapeDtypeStruct((NC * N, 128), jnp.float32),
        mesh=plsc.VectorSubcoreMesh(**_MESH),
        compiler_params=_CPARAMS,
        scratch_types=[
            pltpu.VMEM((_G, K), jnp.int32),
            pltpu.VMEM((_G, K), jnp.int32),
            pltpu.VMEM((_NBUF, K, 128), jnp.float32),
            pltpu.SemaphoreType.DMA((_NBUF,)),
            pltpu.SemaphoreType.DMA((_NBUF,)),
            pltpu.VMEM_SHARED((N + 8, 128), jnp.float32),
        ],
    )(vrows, src2d, dst2d)


_BN = 5000


def _dis(dg_ref):
    return lax.rsqrt(dg_ref[:, 0:1] + dg_ref[:, 1:2] + 1.0)


def _tca_body(x_ref, dg_ref, o_ref):
    o_ref[...] = x_ref[...] * _dis(dg_ref)


def _tca(x, degT):
    return pl.pallas_call(
        _tca_body,
        grid=(N // _BN,),
        in_specs=[
            pl.BlockSpec((_BN, 128), lambda i: (i, 0)),
            pl.BlockSpec((_BN, 2), lambda i: (i, 0)),
        ],
        out_specs=pl.BlockSpec((_BN, 128), lambda i: (i, 0)),
        out_shape=jax.ShapeDtypeStruct((N, 128), jnp.float32),
    )(x, degT)


def _tcb_body(u_ref, xh_ref, dg_ref, w1_ref, b1_ref, w2_ref, o_ref):
    dis = _dis(dg_ref)
    y = u_ref[0] + u_ref[1] + xh_ref[...]
    h = jnp.dot(y, w1_ref[...], preferred_element_type=jnp.float32)
    z = jnp.maximum(h * dis + b1_ref[...], 0.0)
    o_ref[...] = jnp.dot(z * dis, w2_ref[...],
                         preferred_element_type=jnp.float32)


def _tcb(u, xh, degT, W1, b1r, W2):
    return pl.pallas_call(
        _tcb_body,
        grid=(N // _BN,),
        in_specs=[
            pl.BlockSpec((2, _BN, 128), lambda i: (0, i, 0)),
            pl.BlockSpec((_BN, 128), lambda i: (i, 0)),
            pl.BlockSpec((_BN, 2), lambda i: (i, 0)),
            pl.BlockSpec((128, 256), lambda i: (0, 0)),
            pl.BlockSpec((1, 256), lambda i: (0, 0)),
            pl.BlockSpec((256, 128), lambda i: (0, 0)),
        ],
        out_specs=pl.BlockSpec((_BN, 128), lambda i: (i, 0)),
        out_shape=jax.ShapeDtypeStruct((N, 128), jnp.float32),
    )(u, xh, degT, W1, b1r, W2)


def _tcc_body(s_ref, v_ref, dg_ref, b_ref, o_ref):
    o_ref[...] = ((s_ref[0] + s_ref[1] + v_ref[...]) * _dis(dg_ref)
                  + b_ref[...])


def _tcc(s2, v2, degT, b2r):
    return pl.pallas_call(
        _tcc_body,
        grid=(N // _BN,),
        in_specs=[
            pl.BlockSpec((2, _BN, 128), lambda i: (0, i, 0)),
            pl.BlockSpec((_BN, 128), lambda i: (i, 0)),
            pl.BlockSpec((_BN, 2), lambda i: (i, 0)),
            pl.BlockSpec((1, 128), lambda i: (0, 0)),
        ],
        out_specs=pl.BlockSpec((_BN, 128), lambda i: (i, 0)),
        out_shape=jax.ShapeDtypeStruct((N, 128), jnp.float32),
    )(s2, v2, degT, b2r)


def kernel(x, edge_index, W1, b1, W2, b2):
    ei = edge_index.astype(jnp.int32)
    src2d = jnp.concatenate([ei[0], jnp.zeros((PAD,), jnp.int32)]
                            ).reshape(NCH, K)
    pad_dst = N + (jnp.arange(PAD, dtype=jnp.int32) % 8)
    dst2d = jnp.concatenate([ei[1], pad_dst]).reshape(NCH, K)

    degT = _deg_call(dst2d).reshape(NC, DEGW).T
    xh = _tca(x, degT)
    u = _agg_call(xh, src2d, dst2d)
    v2 = _tcb(u.reshape(2, N, 128), xh, degT,
              W1, b1.reshape(1, 256), W2)
    s2 = _agg_call(v2, src2d, dst2d)
    return _tcc(s2.reshape(2, N, 128), v2, degT, b2.reshape(1, 128))
